# baseline (speedup 1.0000x reference)
import functools

import jax
import jax.numpy as jnp
from jax import lax
from jax.experimental import pallas as pl
from jax.experimental.pallas import tpu as pltpu
from jax.experimental.pallas import tpu_sc as plsc

BATCH = 16384

_INFO = plsc.get_sparse_core_info()
NC = _INFO.num_cores
NS = _INFO.num_subcores
L = _INFO.num_lanes
NW = NC * NS
BPW = BATCH // NW

_mesh = plsc.VectorSubcoreMesh(core_axis_name="c", subcore_axis_name="s")


def _split_body(th_ref, t0_ref, t1_ref):
    x = th_ref[...]
    t0_ref[...] = x[:, 0]
    t1_ref[...] = x[:, 1]


def _split_columns(table, block):
    n = table.shape[0]
    grid = pl.cdiv(n, block)
    return pl.pallas_call(
        _split_body,
        grid=(grid,),
        in_specs=[pl.BlockSpec((block, 2), lambda i: (i, 0))],
        out_specs=[
            pl.BlockSpec((block,), lambda i: (i,)),
            pl.BlockSpec((block,), lambda i: (i,)),
        ],
        out_shape=[jax.ShapeDtypeStruct((n,), jnp.float32)] * 2,
    )(table)


@functools.partial(
    pl.kernel,
    mesh=_mesh,
    out_type=jax.ShapeDtypeStruct((BATCH,), jnp.float32),
    scratch_types=[
        pltpu.VMEM((BPW,), jnp.int32),
        pltpu.VMEM((BPW,), jnp.int32),
        pltpu.VMEM((BPW,), jnp.float32),
        pltpu.VMEM((BPW,), jnp.float32),
        pltpu.VMEM((BPW,), jnp.float32),
        pltpu.VMEM((BPW,), jnp.float32),
        pltpu.VMEM((BPW,), jnp.float32),
        pltpu.VMEM((BPW,), jnp.float32),
        pltpu.SemaphoreType.DMA,
    ],
)
def _mirt_sc(stu_hbm, exer_hbm, t0_hbm, t1_hbm, a0_hbm, a1_hbm, be_hbm,
             out_hbm, idx_s, idx_e, t0_v, t1_v, a0_v, a1_v, be_v, out_v, sem):
    wid = lax.axis_index("s") * NC + lax.axis_index("c")
    base = wid * BPW

    pltpu.sync_copy(stu_hbm.at[pl.ds(base, BPW)], idx_s)
    pltpu.sync_copy(exer_hbm.at[pl.ds(base, BPW)], idx_e)

    copies = [
        pltpu.async_copy(t0_hbm.at[idx_s], t0_v, sem),
        pltpu.async_copy(t1_hbm.at[idx_s], t1_v, sem),
        pltpu.async_copy(a0_hbm.at[idx_e], a0_v, sem),
        pltpu.async_copy(a1_hbm.at[idx_e], a1_v, sem),
        pltpu.async_copy(be_hbm.at[idx_e], be_v, sem),
    ]
    for c in copies:
        c.wait()

    for g in range(BPW // L):
        sl = pl.ds(g * L, L)
        t0 = t0_v[sl]
        t1 = t1_v[sl]
        a0 = a0_v[sl]
        a1 = a1_v[sl]
        b = be_v[sl]
        pred = a0 * (t0 - b) + a1 * (t1 - b)
        out_v[sl] = 1.0 / (1.0 + jnp.exp(-pred))

    pltpu.sync_copy(out_v, out_hbm.at[pl.ds(base, BPW)])


def kernel(stu_id, exer_id, theta_table, alpha_table, beta_table):
    t0, t1 = _split_columns(theta_table, 8192)
    a0, a1 = _split_columns(alpha_table, 8192)
    return _mirt_sc(
        stu_id.astype(jnp.int32),
        exer_id.astype(jnp.int32),
        t0,
        t1,
        a0,
        a1,
        jnp.reshape(beta_table, (-1,)),
    )


# speedup vs baseline: 3.3355x; 3.3355x over previous
import functools

import jax
import jax.numpy as jnp
from jax import lax
from jax.experimental import pallas as pl
from jax.experimental.pallas import tpu as pltpu
from jax.experimental.pallas import tpu_sc as plsc
from jax.experimental.layout import Layout, with_layout_constraint

BATCH = 16384
THETA_N = 1000000
ALPHA_N = 100000
TH_NH = (THETA_N // 128) * 128
AL_NH = (ALPHA_N // 128) * 128

_INFO = plsc.get_sparse_core_info()
NC = _INFO.num_cores
NS = _INFO.num_subcores
L = _INFO.num_lanes
NW = NC * NS
BPW = BATCH // NW

_mesh = plsc.VectorSubcoreMesh(core_axis_name="c", subcore_axis_name="s")


def _flat_view(table, nh):
    head = table[:nh]
    b = jnp.reshape(head, (nh // 128, 128, 2))
    b = with_layout_constraint(b, Layout(major_to_minor=(0, 2, 1)))
    bt = jnp.transpose(b, (0, 2, 1))
    bt = with_layout_constraint(bt, Layout(major_to_minor=(0, 1, 2)))
    flat = jnp.reshape(bt, (nh * 2,))
    return flat, table[nh:, 0], table[nh:, 1]


@functools.partial(
    pl.kernel,
    mesh=_mesh,
    out_type=jax.ShapeDtypeStruct((BATCH,), jnp.float32),
    scratch_types=[
        pltpu.VMEM((BPW,), jnp.int32),
        pltpu.VMEM((BPW,), jnp.int32),
        pltpu.VMEM((BPW,), jnp.int32),
        pltpu.VMEM((BPW,), jnp.int32),
        pltpu.VMEM((BPW,), jnp.int32),
        pltpu.VMEM((BPW,), jnp.int32),
        pltpu.VMEM((BPW,), jnp.int32),
        pltpu.VMEM((BPW,), jnp.int32),
        pltpu.VMEM((BPW,), jnp.float32),
        pltpu.VMEM((BPW,), jnp.float32),
        pltpu.VMEM((BPW,), jnp.float32),
        pltpu.VMEM((BPW,), jnp.float32),
        pltpu.VMEM((BPW,), jnp.float32),
        pltpu.VMEM((BPW,), jnp.float32),
        pltpu.VMEM((BPW,), jnp.float32),
        pltpu.VMEM((BPW,), jnp.float32),
        pltpu.VMEM((BPW,), jnp.float32),
        pltpu.VMEM((BPW,), jnp.float32),
        pltpu.SemaphoreType.DMA,
    ],
)
def _mirt_sc(stu_hbm, exer_hbm, thf_hbm, tht0_hbm, tht1_hbm,
             alf_hbm, alt0_hbm, alt1_hbm, be_hbm, out_hbm,
             idx_s, idx_e, ix_t0, ix_t1, ix_tt, ix_a0, ix_a1, ix_at,
             t0m_v, t1m_v, t0t_v, t1t_v, a0m_v, a1m_v, a0t_v, a1t_v,
             be_v, out_v, sem):
    wid = lax.axis_index("s") * NC + lax.axis_index("c")
    base = wid * BPW

    pltpu.sync_copy(stu_hbm.at[pl.ds(base, BPW)], idx_s)
    pltpu.sync_copy(exer_hbm.at[pl.ds(base, BPW)], idx_e)

    for g in range(BPW // L):
        sl = pl.ds(g * L, L)
        i = idx_s[sl]
        blk = lax.shift_right_logical(i, 7)
        off = lax.bitwise_and(i, 127)
        in_main = i < TH_NH
        a0 = lax.shift_left(blk, 8) + off
        ix_t0[sl] = jnp.where(in_main, a0, 0)
        ix_t1[sl] = jnp.where(in_main, a0 + 128, 128)
        ix_tt[sl] = jnp.where(in_main, lax.bitwise_and(off, 63), i - TH_NH)
        j = idx_e[sl]
        jblk = lax.shift_right_logical(j, 7)
        joff = lax.bitwise_and(j, 127)
        jin = j < AL_NH
        ja0 = lax.shift_left(jblk, 8) + joff
        ix_a0[sl] = jnp.where(jin, ja0, 0)
        ix_a1[sl] = jnp.where(jin, ja0 + 128, 128)
        ix_at[sl] = jnp.where(jin, lax.bitwise_and(joff, 31), j - AL_NH)

    copies = [
        pltpu.async_copy(thf_hbm.at[ix_t0], t0m_v, sem),
        pltpu.async_copy(thf_hbm.at[ix_t1], t1m_v, sem),
        pltpu.async_copy(tht0_hbm.at[ix_tt], t0t_v, sem),
        pltpu.async_copy(tht1_hbm.at[ix_tt], t1t_v, sem),
        pltpu.async_copy(alf_hbm.at[ix_a0], a0m_v, sem),
        pltpu.async_copy(alf_hbm.at[ix_a1], a1m_v, sem),
        pltpu.async_copy(alt0_hbm.at[ix_at], a0t_v, sem),
        pltpu.async_copy(alt1_hbm.at[ix_at], a1t_v, sem),
        pltpu.async_copy(be_hbm.at[idx_e], be_v, sem),
    ]
    for c in copies:
        c.wait()

    for g in range(BPW // L):
        sl = pl.ds(g * L, L)
        in_main = idx_s[sl] < TH_NH
        jin = idx_e[sl] < AL_NH
        t0 = jnp.where(in_main, t0m_v[sl], t0t_v[sl])
        t1 = jnp.where(in_main, t1m_v[sl], t1t_v[sl])
        a0 = jnp.where(jin, a0m_v[sl], a0t_v[sl])
        a1 = jnp.where(jin, a1m_v[sl], a1t_v[sl])
        b = be_v[sl]
        pred = a0 * (t0 - b) + a1 * (t1 - b)
        out_v[sl] = 1.0 / (1.0 + jnp.exp(-pred))

    pltpu.sync_copy(out_v, out_hbm.at[pl.ds(base, BPW)])


def kernel(stu_id, exer_id, theta_table, alpha_table, beta_table):
    thf, tht0, tht1 = _flat_view(theta_table, TH_NH)
    alf, alt0, alt1 = _flat_view(alpha_table, AL_NH)
    return _mirt_sc(
        stu_id.astype(jnp.int32),
        exer_id.astype(jnp.int32),
        thf,
        tht0,
        tht1,
        alf,
        alt0,
        alt1,
        jnp.reshape(beta_table, (-1,)),
    )


# speedup vs baseline: 22.3699x; 6.7065x over previous
import functools

import jax
import jax.numpy as jnp
from jax import lax
from jax.experimental import pallas as pl
from jax.experimental.pallas import tpu as pltpu
from jax.experimental.pallas import tpu_sc as plsc
from jax.experimental.layout import Layout, with_layout_constraint

BATCH = 16384
THETA_N = 1000000
ALPHA_N = 100000
TH_NH = (THETA_N // 128) * 128
AL_NH = (ALPHA_N // 128) * 128

_INFO = plsc.get_sparse_core_info()
NC = _INFO.num_cores
NS = _INFO.num_subcores
L = _INFO.num_lanes
NW = NC * NS
BPW = BATCH // NW

_mesh = plsc.VectorSubcoreMesh(core_axis_name="c", subcore_axis_name="s")


def _take(x, i):
    return x.at[i].get(mode="promise_in_bounds")


def _flat_view(table, nh):
    head = table[:nh]
    b = jnp.reshape(head, (nh // 128, 128, 2))
    b = with_layout_constraint(b, Layout(major_to_minor=(0, 2, 1)))
    bt = jnp.transpose(b, (0, 2, 1))
    bt = with_layout_constraint(bt, Layout(major_to_minor=(0, 1, 2)))
    flat = jnp.reshape(bt, (nh * 2,))
    return flat, table[nh:, 0], table[nh:, 1]


@functools.partial(
    pl.kernel,
    mesh=_mesh,
    out_type=jax.ShapeDtypeStruct((BATCH,), jnp.float32),
    scratch_types=[
        pltpu.VMEM((BPW,), jnp.int32),
        pltpu.VMEM((BPW,), jnp.int32),
        pltpu.VMEM((BPW,), jnp.int32),
        pltpu.VMEM((BPW,), jnp.int32),
        pltpu.VMEM((BPW,), jnp.int32),
        pltpu.VMEM((BPW,), jnp.int32),
        pltpu.VMEM((BPW,), jnp.float32),
        pltpu.VMEM((BPW,), jnp.float32),
        pltpu.VMEM((BPW,), jnp.float32),
        pltpu.VMEM((BPW,), jnp.float32),
        pltpu.VMEM((64,), jnp.float32),
        pltpu.VMEM((64,), jnp.float32),
        pltpu.VMEM((32,), jnp.float32),
        pltpu.VMEM((32,), jnp.float32),
        pltpu.VMEM((BPW,), jnp.float32),
        pltpu.VMEM((BPW,), jnp.float32),
        pltpu.SemaphoreType.DMA,
    ],
)
def _mirt_sc(stu_hbm, exer_hbm, thf_hbm, tht0_hbm, tht1_hbm,
             alf_hbm, alt0_hbm, alt1_hbm, be_hbm, out_hbm,
             idx_s, idx_e, ix_t0, ix_t1, ix_a0, ix_a1,
             t0m_v, t1m_v, a0m_v, a1m_v, tt0_v, tt1_v, at0_v, at1_v,
             be_v, out_v, sem):
    wid = lax.axis_index("s") * NC + lax.axis_index("c")
    base = wid * BPW

    pltpu.sync_copy(stu_hbm.at[pl.ds(base, BPW)], idx_s)
    pltpu.sync_copy(exer_hbm.at[pl.ds(base, BPW)], idx_e)
    pltpu.sync_copy(tht0_hbm, tt0_v)
    pltpu.sync_copy(tht1_hbm, tt1_v)
    pltpu.sync_copy(alt0_hbm, at0_v)
    pltpu.sync_copy(alt1_hbm, at1_v)

    for g in range(BPW // L):
        sl = pl.ds(g * L, L)
        i = idx_s[sl]
        blk = lax.shift_right_logical(i, 7)
        off = lax.bitwise_and(i, 127)
        in_main = i < TH_NH
        a0 = lax.shift_left(blk, 8) + off
        ix_t0[sl] = jnp.where(in_main, a0, 0)
        ix_t1[sl] = jnp.where(in_main, a0 + 128, 128)
        j = idx_e[sl]
        jblk = lax.shift_right_logical(j, 7)
        joff = lax.bitwise_and(j, 127)
        jin = j < AL_NH
        ja0 = lax.shift_left(jblk, 8) + joff
        ix_a0[sl] = jnp.where(jin, ja0, 0)
        ix_a1[sl] = jnp.where(jin, ja0 + 128, 128)

    copies = [
        pltpu.async_copy(thf_hbm.at[ix_t0], t0m_v, sem),
        pltpu.async_copy(thf_hbm.at[ix_t1], t1m_v, sem),
        pltpu.async_copy(alf_hbm.at[ix_a0], a0m_v, sem),
        pltpu.async_copy(alf_hbm.at[ix_a1], a1m_v, sem),
        pltpu.async_copy(be_hbm.at[idx_e], be_v, sem),
    ]
    for c in copies:
        c.wait()

    tt0 = [tt0_v[pl.ds(k * L, L)] for k in range(4)]
    tt1 = [tt1_v[pl.ds(k * L, L)] for k in range(4)]
    at0 = [at0_v[pl.ds(k * L, L)] for k in range(2)]
    at1 = [at1_v[pl.ds(k * L, L)] for k in range(2)]

    def _tail_pick(vregs, tidx):
        sel = lax.bitwise_and(tidx, 15)
        hi = lax.shift_right_logical(tidx, 4)
        v = _take(vregs[0], sel)
        for k in range(1, len(vregs)):
            v = jnp.where(hi == k, _take(vregs[k], sel), v)
        return v

    for g in range(BPW // L):
        sl = pl.ds(g * L, L)
        i = idx_s[sl]
        j = idx_e[sl]
        in_main = i < TH_NH
        jin = j < AL_NH
        ti = jnp.where(in_main, 0, i - TH_NH)
        tj = jnp.where(jin, 0, j - AL_NH)
        t0 = jnp.where(in_main, t0m_v[sl], _tail_pick(tt0, ti))
        t1 = jnp.where(in_main, t1m_v[sl], _tail_pick(tt1, ti))
        a0 = jnp.where(jin, a0m_v[sl], _tail_pick(at0, tj))
        a1 = jnp.where(jin, a1m_v[sl], _tail_pick(at1, tj))
        b = be_v[sl]
        pred = a0 * (t0 - b) + a1 * (t1 - b)
        out_v[sl] = 1.0 / (1.0 + jnp.exp(-pred))

    pltpu.sync_copy(out_v, out_hbm.at[pl.ds(base, BPW)])


def kernel(stu_id, exer_id, theta_table, alpha_table, beta_table):
    thf, tht0, tht1 = _flat_view(theta_table, TH_NH)
    alf, alt0, alt1 = _flat_view(alpha_table, AL_NH)
    return _mirt_sc(
        stu_id.astype(jnp.int32),
        exer_id.astype(jnp.int32),
        thf,
        tht0,
        tht1,
        alf,
        alt0,
        alt1,
        jnp.reshape(beta_table, (-1,)),
    )


# speedup vs baseline: 23.9134x; 1.0690x over previous
import functools

import jax
import jax.numpy as jnp
from jax import lax
from jax.experimental import pallas as pl
from jax.experimental.pallas import tpu as pltpu
from jax.experimental.pallas import tpu_sc as plsc
from jax.experimental.layout import Layout, with_layout_constraint

BATCH = 16384
THETA_N = 1000000
ALPHA_N = 100000
TH_NH = (THETA_N // 128) * 128
AL_NH = (ALPHA_N // 128) * 128

_INFO = plsc.get_sparse_core_info()
NC = _INFO.num_cores
NS = _INFO.num_subcores
L = _INFO.num_lanes
NW = NC * NS
BPW = BATCH // NW

_mesh = plsc.VectorSubcoreMesh(core_axis_name="c", subcore_axis_name="s")


_DNUMS = lax.GatherDimensionNumbers(
    offset_dims=(), collapsed_slice_dims=(0,), start_index_map=(0,))


def _take(x, i):
    return lax.gather(x, i[:, None], _DNUMS, (1,),
                      mode=lax.GatherScatterMode.PROMISE_IN_BOUNDS)


def _flat_view(table, nh):
    head = table[:nh]
    b = jnp.reshape(head, (nh // 128, 128, 2))
    b = with_layout_constraint(b, Layout(major_to_minor=(0, 2, 1)))
    bt = jnp.transpose(b, (0, 2, 1))
    bt = with_layout_constraint(bt, Layout(major_to_minor=(0, 1, 2)))
    flat = jnp.reshape(bt, (nh * 2,))
    return flat, table[nh:, 0], table[nh:, 1]


@functools.partial(
    pl.kernel,
    mesh=_mesh,
    out_type=jax.ShapeDtypeStruct((BATCH,), jnp.float32),
    scratch_types=[
        pltpu.VMEM((BPW,), jnp.int32),
        pltpu.VMEM((BPW,), jnp.int32),
        pltpu.VMEM((BPW,), jnp.int32),
        pltpu.VMEM((BPW,), jnp.int32),
        pltpu.VMEM((BPW,), jnp.int32),
        pltpu.VMEM((BPW,), jnp.int32),
        pltpu.VMEM((BPW,), jnp.float32),
        pltpu.VMEM((BPW,), jnp.float32),
        pltpu.VMEM((BPW,), jnp.float32),
        pltpu.VMEM((BPW,), jnp.float32),
        pltpu.VMEM((64,), jnp.float32),
        pltpu.VMEM((64,), jnp.float32),
        pltpu.VMEM((32,), jnp.float32),
        pltpu.VMEM((32,), jnp.float32),
        pltpu.VMEM((BPW,), jnp.float32),
        pltpu.VMEM((BPW,), jnp.float32),
        pltpu.SemaphoreType.DMA,
        pltpu.SemaphoreType.DMA,
    ],
)
def _mirt_sc(stu_hbm, exer_hbm, thf_hbm, tht0_hbm, tht1_hbm,
             alf_hbm, alt0_hbm, alt1_hbm, be_hbm, out_hbm,
             idx_s, idx_e, ix_t0, ix_t1, ix_a0, ix_a1,
             t0m_v, t1m_v, a0m_v, a1m_v, tt0_v, tt1_v, at0_v, at1_v,
             be_v, out_v, sem, sem2):
    wid = lax.axis_index("s") * NC + lax.axis_index("c")
    base = wid * BPW

    stage = [
        pltpu.async_copy(stu_hbm.at[pl.ds(base, BPW)], idx_s, sem2),
        pltpu.async_copy(exer_hbm.at[pl.ds(base, BPW)], idx_e, sem2),
        pltpu.async_copy(tht0_hbm, tt0_v, sem2),
        pltpu.async_copy(tht1_hbm, tt1_v, sem2),
        pltpu.async_copy(alt0_hbm, at0_v, sem2),
        pltpu.async_copy(alt1_hbm, at1_v, sem2),
    ]
    for c in stage:
        c.wait()

    for g in range(BPW // L):
        sl = pl.ds(g * L, L)
        i = idx_s[sl]
        blk = lax.shift_right_logical(i, 7)
        off = lax.bitwise_and(i, 127)
        in_main = i < TH_NH
        a0 = lax.shift_left(blk, 8) + off
        ix_t0[sl] = jnp.where(in_main, a0, 0)
        ix_t1[sl] = jnp.where(in_main, a0 + 128, 128)
        j = idx_e[sl]
        jblk = lax.shift_right_logical(j, 7)
        joff = lax.bitwise_and(j, 127)
        jin = j < AL_NH
        ja0 = lax.shift_left(jblk, 8) + joff
        ix_a0[sl] = jnp.where(jin, ja0, 0)
        ix_a1[sl] = jnp.where(jin, ja0 + 128, 128)

    copies = [
        pltpu.async_copy(thf_hbm.at[ix_t0], t0m_v, sem),
        pltpu.async_copy(thf_hbm.at[ix_t1], t1m_v, sem),
        pltpu.async_copy(alf_hbm.at[ix_a0], a0m_v, sem),
        pltpu.async_copy(alf_hbm.at[ix_a1], a1m_v, sem),
        pltpu.async_copy(be_hbm.at[idx_e], be_v, sem),
    ]
    for c in copies:
        c.wait()

    tt0 = [tt0_v[pl.ds(k * L, L)] for k in range(4)]
    tt1 = [tt1_v[pl.ds(k * L, L)] for k in range(4)]
    at0 = [at0_v[pl.ds(k * L, L)] for k in range(2)]
    at1 = [at1_v[pl.ds(k * L, L)] for k in range(2)]

    def _tail_pick(vregs, tidx):
        sel = lax.bitwise_and(tidx, 15)
        hi = lax.shift_right_logical(tidx, 4)
        v = _take(vregs[0], sel)
        for k in range(1, len(vregs)):
            v = jnp.where(hi == k, _take(vregs[k], sel), v)
        return v

    for g in range(BPW // L):
        sl = pl.ds(g * L, L)
        i = idx_s[sl]
        j = idx_e[sl]
        in_main = i < TH_NH
        jin = j < AL_NH
        ti = jnp.where(in_main, 0, i - TH_NH)
        tj = jnp.where(jin, 0, j - AL_NH)
        t0 = jnp.where(in_main, t0m_v[sl], _tail_pick(tt0, ti))
        t1 = jnp.where(in_main, t1m_v[sl], _tail_pick(tt1, ti))
        a0 = jnp.where(jin, a0m_v[sl], _tail_pick(at0, tj))
        a1 = jnp.where(jin, a1m_v[sl], _tail_pick(at1, tj))
        b = be_v[sl]
        pred = a0 * (t0 - b) + a1 * (t1 - b)
        out_v[sl] = 1.0 / (1.0 + jnp.exp(-pred))

    pltpu.sync_copy(out_v, out_hbm.at[pl.ds(base, BPW)])


def kernel(stu_id, exer_id, theta_table, alpha_table, beta_table):
    thf, tht0, tht1 = _flat_view(theta_table, TH_NH)
    alf, alt0, alt1 = _flat_view(alpha_table, AL_NH)
    return _mirt_sc(
        stu_id.astype(jnp.int32),
        exer_id.astype(jnp.int32),
        thf,
        tht0,
        tht1,
        alf,
        alt0,
        alt1,
        jnp.reshape(beta_table, (-1,)),
    )
